# baseline (speedup 1.0000x reference)
import jax
import jax.numpy as jnp
from jax import lax
from jax.experimental import pallas as pl
from jax.experimental.pallas import tpu as pltpu
from jax.experimental.pallas import tpu_sc as plsc

VOCAB = 100000
DIM = 128
BATCH = 16384
CTX = 20
NEG = 50
NOUTP = NEG + 2

NC = 2
NS = 16
NW = NC * NS
B_PER_W = BATCH // NW
NB = 8
NH = NB // 2
CHUNKS = B_PER_W // NB
LANES = 16
DREGS = DIM // LANES

CTX_ROW_W = 80
WO_ROW_W = 104
HROWS = NH * NOUTP

PART = BATCH * NOUTP * LANES
TC_ROWS = PART // DIM
TC_BLOCK = 8192
TC_GRID = TC_ROWS // TC_BLOCK


def _sc_scores(ctx_idx_hbm, wo_idx_hbm, w_in_hbm, w_out_hbm,
               part_out_hbm,
               ctx_i, wo_i, ctx_r_v, wo_h0, wo_h1, part0, part1,
               isem, csem, wsa, wsb, osa, osb):
    wid = lax.axis_index("s") * NC + lax.axis_index("c")
    wo_h = (wo_h0, wo_h1)
    parts = (part0, part1)
    wsem = (wsa, wsb)
    osem = (osa, osb)

    def start_idx(c):
        gc = wid * CHUNKS + c
        pltpu.async_copy(ctx_idx_hbm.at[pl.ds(gc * 2, 2)], ctx_i, isem)
        pltpu.async_copy(wo_idx_hbm.at[pl.ds(gc * 4, 4)], wo_i, isem)

    def wait_idx(c):
        gc = wid * CHUNKS + c
        pltpu.make_async_copy(ctx_idx_hbm.at[pl.ds(gc * 2, 2)],
                              ctx_i, isem).wait()
        pltpu.make_async_copy(wo_idx_hbm.at[pl.ds(gc * 4, 4)],
                              wo_i, isem).wait()

    def fire_ctx():
        for j in range(2):
            pltpu.async_copy(
                w_in_hbm.at[ctx_i.at[j]],
                ctx_r_v.at[pl.ds(j * CTX_ROW_W, CTX_ROW_W)], csem)

    def wait_ctx():
        for j in range(2):
            pltpu.make_async_copy(
                w_in_hbm.at[ctx_i.at[j]],
                ctx_r_v.at[pl.ds(j * CTX_ROW_W, CTX_ROW_W)], csem).wait()

    def fire_half(hf):
        for j in range(2):
            pltpu.async_copy(
                w_out_hbm.at[wo_i.at[2 * hf + j]],
                wo_h[hf].at[pl.ds(j * WO_ROW_W, WO_ROW_W)], wsem[hf])

    def wait_half(hf):
        for j in range(2):
            pltpu.make_async_copy(
                w_out_hbm.at[wo_i.at[2 * hf + j]],
                wo_h[hf].at[pl.ds(j * WO_ROW_W, WO_ROW_W)],
                wsem[hf]).wait()

    def start_out(c, hf):
        gc = wid * CHUNKS + c
        pltpu.async_copy(
            parts[hf],
            part_out_hbm.at[pl.ds(gc * 2 * HROWS + hf * HROWS, HROWS)],
            osem[hf])

    def wait_out(c, hf):
        gc = wid * CHUNKS + c
        pltpu.make_async_copy(
            parts[hf],
            part_out_hbm.at[pl.ds(gc * 2 * HROWS + hf * HROWS, HROWS)],
            osem[hf]).wait()

    def compute_half(hf):
        wo_v = wo_h[hf]
        part_v = parts[hf]

        @plsc.parallel_loop(0, NH)
        def b_body(b):
            r0 = (hf * NH + b) * CTX
            h0 = tuple(ctx_r_v[r0, pl.ds(d * LANES, LANES)]
                       for d in range(DREGS))

            def c_body(c, h):
                return tuple(
                    h[d] + ctx_r_v[r0 + c, pl.ds(d * LANES, LANES)]
                    for d in range(DREGS))

            h = lax.fori_loop(1, CTX, c_body, h0, unroll=5)
            h = tuple(hd * (1.0 / CTX) for hd in h)

            o0 = b * NOUTP

            @plsc.parallel_loop(0, NOUTP, unroll=4)
            def k_body(k):
                row = o0 + k
                p = [wo_v[row, pl.ds(d * LANES, LANES)] * h[d]
                     for d in range(DREGS)]
                acc = ((p[0] + p[1]) + (p[2] + p[3])) + \
                      ((p[4] + p[5]) + (p[6] + p[7]))
                part_v[row] = acc

    start_idx(0)
    wait_idx(0)
    fire_ctx()
    fire_half(0)
    fire_half(1)

    def chunk_body(c, carry):
        wait_ctx()
        wait_half(0)

        @pl.when(c > 0)
        def _():
            wait_out(c - 1, 0)

        compute_half(0)
        start_out(c, 0)
        wait_half(1)

        @pl.when(c > 0)
        def _():
            wait_out(c - 1, 1)

        @pl.when(c < CHUNKS - 1)
        def _():
            start_idx(c + 1)
            wait_idx(c + 1)
            fire_half(0)

        compute_half(1)
        start_out(c, 1)

        @pl.when(c < CHUNKS - 1)
        def _():
            fire_ctx()
            fire_half(1)

        return carry

    lax.fori_loop(0, CHUNKS, chunk_body, 0)
    wait_out(CHUNKS - 1, 0)
    wait_out(CHUNKS - 1, 1)


_sc_call = pl.kernel(
    _sc_scores,
    out_type=jax.ShapeDtypeStruct((PART // LANES, LANES), jnp.float32),
    mesh=plsc.VectorSubcoreMesh(core_axis_name="c", subcore_axis_name="s"),
    scratch_types=[
        pltpu.VMEM((2, CTX_ROW_W), jnp.int32),
        pltpu.VMEM((4, WO_ROW_W), jnp.int32),
        pltpu.VMEM((NB * CTX, DIM), jnp.float32),
        pltpu.VMEM((HROWS, DIM), jnp.float32),
        pltpu.VMEM((HROWS, DIM), jnp.float32),
        pltpu.VMEM((HROWS, LANES), jnp.float32),
        pltpu.VMEM((HROWS, LANES), jnp.float32),
        pltpu.SemaphoreType.DMA,
        pltpu.SemaphoreType.DMA,
        pltpu.SemaphoreType.DMA,
        pltpu.SemaphoreType.DMA,
        pltpu.SemaphoreType.DMA,
        pltpu.SemaphoreType.DMA,
    ],
    compiler_params=pltpu.CompilerParams(needs_layout_passes=False),
)


def _softplus(x):
    return jnp.maximum(x, 0.0) + jnp.log1p(jnp.exp(-jnp.abs(x)))


def _loss_body(part_ref, out_ref):
    pid = pl.program_id(0)
    x = part_ref[...]
    gi = lax.broadcasted_iota(jnp.int32, (DIM, DIM // LANES), 0) // LANES
    gj = lax.broadcasted_iota(jnp.int32, (DIM, DIM // LANES), 1)
    fold = (gi == gj).astype(jnp.float32)
    s = jax.lax.dot(x, fold, precision=jax.lax.Precision.HIGHEST)
    r = lax.broadcasted_iota(jnp.int32, s.shape, 0) + pid * TC_BLOCK
    c = lax.broadcasted_iota(jnp.int32, s.shape, 1)
    k = (r * (DIM // LANES) + c) % NOUTP
    val = jnp.where(k < NEG, _softplus(s),
                    jnp.where(k == NEG, _softplus(-s), 0.0))

    @pl.when(pid == 0)
    def _():
        out_ref[0, 0] = 0.0

    out_ref[0, 0] += jnp.sum(val) * (1.0 / BATCH)


_loss_call = pl.pallas_call(
    _loss_body,
    grid=(TC_GRID,),
    in_specs=[pl.BlockSpec((TC_BLOCK, DIM), lambda i: (i, 0))],
    out_shape=jax.ShapeDtypeStruct((1, 1), jnp.float32),
    out_specs=pl.BlockSpec(memory_space=pltpu.SMEM),
)


def kernel(context_words, target_words, negative_words, W_in, W_out):
    ctx = context_words.astype(jnp.int32).reshape(
        BATCH * CTX // CTX_ROW_W, CTX_ROW_W)
    tgt = target_words.astype(jnp.int32)[:, None]
    wo = jnp.concatenate(
        [negative_words.astype(jnp.int32), tgt, tgt], axis=1).reshape(
        BATCH * NOUTP // WO_ROW_W, WO_ROW_W)
    part = _sc_call(ctx, wo, W_in, W_out)
    loss = _loss_call(part.reshape(TC_ROWS, DIM))
    return loss[0, 0]


# speedup vs baseline: 1.2723x; 1.2723x over previous
import jax
import jax.numpy as jnp
from jax import lax
from jax.experimental import pallas as pl
from jax.experimental.pallas import tpu as pltpu
from jax.experimental.pallas import tpu_sc as plsc

VOCAB = 100000
DIM = 128
BATCH = 16384
CTX = 20
NEG = 50
NOUT = NEG + 1
NOUTP = NOUT + 1

NC = 2
NS = 16
NW = NC * NS
B_PER_W = BATCH // NW
NB = 8
CHUNKS = B_PER_W // NB
LANES = 16
DREGS = DIM // LANES

CTX_IDX_ROW = 80
WO_IDX_ROW = 102
WO_ROWS = NB * NOUT

PART = BATCH * NOUTP * LANES
TC_ROWS = PART // DIM
TC_BLOCK = 8192
TC_GRID = TC_ROWS // TC_BLOCK


def _sc_scores(ctx_idx_hbm, wo_idx_hbm, w_in_hbm, w_out_hbm,
               part_out_hbm,
               ctx_i_v, wo_i_v, ctx_r_v, wo_r_v, part_v, sem):
    wid = lax.axis_index("s") * NC + lax.axis_index("c")

    def chunk_body(chunk, carry):
        g = wid * CHUNKS + chunk
        base = g * NB

        pltpu.sync_copy(ctx_idx_hbm.at[pl.ds(g * 2, 2)], ctx_i_v)
        pltpu.sync_copy(wo_idx_hbm.at[pl.ds(g * 4, 4)], wo_i_v)

        cps = []
        for j in range(2):
            cps.append(pltpu.async_copy(
                w_in_hbm.at[ctx_i_v.at[j]],
                ctx_r_v.at[pl.ds(j * CTX_IDX_ROW, CTX_IDX_ROW)], sem))
        for j in range(4):
            cps.append(pltpu.async_copy(
                w_out_hbm.at[wo_i_v.at[j]],
                wo_r_v.at[pl.ds(j * WO_IDX_ROW, WO_IDX_ROW)], sem))
        for cp in cps:
            cp.wait()

        @plsc.parallel_loop(0, NB)
        def b_body(b):
            r0 = b * CTX
            h0 = tuple(ctx_r_v[r0, pl.ds(j * LANES, LANES)]
                       for j in range(DREGS))

            def c_body(c, h):
                return tuple(h[j] + ctx_r_v[r0 + c, pl.ds(j * LANES, LANES)]
                             for j in range(DREGS))

            h = lax.fori_loop(1, CTX, c_body, h0, unroll=5)
            h = tuple(hj * (1.0 / CTX) for hj in h)

            nr0 = b * NOUT
            o0 = b * NOUTP * LANES

            @plsc.parallel_loop(0, NOUTP, unroll=4)
            def k_body(k):
                row = jnp.minimum(nr0 + k, WO_ROWS - 1)
                p = [wo_r_v[row, pl.ds(j * LANES, LANES)] * h[j]
                     for j in range(DREGS)]
                acc = ((p[0] + p[1]) + (p[2] + p[3])) + \
                      ((p[4] + p[5]) + (p[6] + p[7]))
                part_v[pl.ds(o0 + k * LANES, LANES)] = acc

        pltpu.sync_copy(
            part_v,
            part_out_hbm.at[pl.ds(base * NOUTP * LANES, NB * NOUTP * LANES)])
        return carry

    lax.fori_loop(0, CHUNKS, chunk_body, 0)


_sc_call = pl.kernel(
    _sc_scores,
    out_type=jax.ShapeDtypeStruct((PART,), jnp.float32),
    mesh=plsc.VectorSubcoreMesh(core_axis_name="c", subcore_axis_name="s"),
    scratch_types=[
        pltpu.VMEM((2, CTX_IDX_ROW), jnp.int32),
        pltpu.VMEM((4, WO_IDX_ROW), jnp.int32),
        pltpu.VMEM((NB * CTX, DIM), jnp.float32),
        pltpu.VMEM((WO_ROWS, DIM), jnp.float32),
        pltpu.VMEM((NB * NOUTP * LANES,), jnp.float32),
        pltpu.SemaphoreType.DMA,
    ],
    compiler_params=pltpu.CompilerParams(needs_layout_passes=False),
)


def _softplus(x):
    return jnp.maximum(x, 0.0) + jnp.log1p(jnp.exp(-jnp.abs(x)))


def _loss_body(part_ref, out_ref):
    pid = pl.program_id(0)
    x = part_ref[...]
    gi = lax.broadcasted_iota(jnp.int32, (DIM, DIM // LANES), 0) // LANES
    gj = lax.broadcasted_iota(jnp.int32, (DIM, DIM // LANES), 1)
    fold = (gi == gj).astype(jnp.float32)
    s = jax.lax.dot(x, fold, precision=jax.lax.Precision.HIGHEST)
    r = lax.broadcasted_iota(jnp.int32, s.shape, 0) + pid * TC_BLOCK
    c = lax.broadcasted_iota(jnp.int32, s.shape, 1)
    k = (r * (DIM // LANES) + c) % NOUTP
    val = jnp.where(k < NEG, _softplus(s),
                    jnp.where(k == NEG, _softplus(-s), 0.0))

    @pl.when(pid == 0)
    def _():
        out_ref[0, 0] = 0.0

    out_ref[0, 0] += jnp.sum(val) * (1.0 / BATCH)


_loss_call = pl.pallas_call(
    _loss_body,
    grid=(TC_GRID,),
    in_specs=[pl.BlockSpec((TC_BLOCK, DIM), lambda i: (i, 0))],
    out_shape=jax.ShapeDtypeStruct((1, 1), jnp.float32),
    out_specs=pl.BlockSpec(memory_space=pltpu.SMEM),
)


def kernel(context_words, target_words, negative_words, W_in, W_out):
    ctx = context_words.astype(jnp.int32).reshape(
        BATCH * CTX // CTX_IDX_ROW, CTX_IDX_ROW)
    wo = jnp.concatenate(
        [negative_words.astype(jnp.int32),
         target_words.astype(jnp.int32)[:, None]], axis=1).reshape(
        BATCH * NOUT // WO_IDX_ROW, WO_IDX_ROW)
    part = _sc_call(ctx, wo, W_in, W_out)
    loss = _loss_call(part.reshape(TC_ROWS, DIM))
    return loss[0, 0]
